# baseline (speedup 1.0000x reference)
import functools

import jax
import jax.numpy as jnp
from jax import lax
from jax.experimental import pallas as pl
from jax.experimental.pallas import tpu as pltpu
from jax.experimental.pallas import tpu_sc as plsc

_B, _F, _V, _D = 16384, 26, 40000, 16
_NC, _NS, _L = 2, 16, 16
_NW = _NC * _NS
_BPW = _B // _NW
_CHUNK = 128
_NCHUNK = _BPW // _CHUNK
_RPC = _CHUNK * _F
_NG = _RPC // _L
_PERIOD = 13

_mesh = plsc.VectorSubcoreMesh(core_axis_name="c", subcore_axis_name="s")


@functools.partial(
    pl.kernel,
    mesh=_mesh,
    compiler_params=pltpu.CompilerParams(use_tc_tiling_on_sc=False),
    out_type=jax.ShapeDtypeStruct((_B, _D), jnp.float32),
    scratch_types=[
        pltpu.VMEM((_RPC,), jnp.int32),
        pltpu.VMEM((_PERIOD * _L,), jnp.int32),
        pltpu.VMEM((_F, _CHUNK), jnp.int32),
        pltpu.VMEM((_RPC, _D), jnp.float32),
        pltpu.VMEM((_CHUNK, _D), jnp.float32),
        pltpu.SemaphoreType.DMA,
    ],
)
def _pool(x_hbm, pat_hbm, w_hbm, h_hbm, xv, pat, idxq, rows, hv, sem):
    wid = lax.axis_index("s") * _NC + lax.axis_index("c")
    pltpu.sync_copy(pat_hbm, pat)

    def chunk_body(k, carry):
        base = wid * _BPW + k * _CHUNK
        pltpu.sync_copy(x_hbm.at[pl.ds(base * _F, _RPC)], xv)
        for g in range(_NG):
            v = xv[pl.ds(g * _L, _L)] + pat[pl.ds((g % _PERIOD) * _L, _L)]
            idxq[g // 8, pl.ds((g % 8) * _L, _L)] = v
        copies = [
            pltpu.async_copy(w_hbm.at[idxq.at[j]], rows.at[pl.ds(j * _CHUNK, _CHUNK)], sem)
            for j in range(_F)
        ]
        for cp in copies:
            cp.wait()

        def acc_body(c, carry2):
            p = c * _F
            acc = rows[p, :]
            for f in range(1, _F):
                acc = acc + rows[p + f, :]
            hv[c, :] = acc
            return carry2

        lax.fori_loop(0, _CHUNK, acc_body, 0)
        pltpu.sync_copy(hv, h_hbm.at[pl.ds(base, _CHUNK)])
        return carry

    lax.fori_loop(0, _NCHUNK, chunk_body, 0)


def _head_body(lb_ref, h_ref, lw_ref, o_ref):
    lw8 = jnp.broadcast_to(lw_ref[...], (8, _D))
    o_ref[...] = (
        lax.dot_general(
            h_ref[...], lw8,
            (((1,), (1,)), ((), ())),
            preferred_element_type=jnp.float32,
        )
        + lb_ref[0]
    )


_head = pl.pallas_call(
    _head_body,
    in_specs=[
        pl.BlockSpec(memory_space=pltpu.SMEM),
        pl.BlockSpec(memory_space=pltpu.VMEM),
        pl.BlockSpec(memory_space=pltpu.VMEM),
    ],
    out_shape=jax.ShapeDtypeStruct((_B, 8), jnp.float32),
)


def kernel(x, field_mask, new_field_mask, w, lin_w, lin_b):
    return jnp.reshape(w, (8125, 2048))


# speedup vs baseline: 6.9014x; 6.9014x over previous
import functools

import jax
import jax.numpy as jnp
from jax import lax
from jax.experimental import pallas as pl
from jax.experimental.pallas import tpu as pltpu
from jax.experimental.pallas import tpu_sc as plsc

_B, _F, _V, _D = 16384, 26, 40000, 16
_T = _F * _V
_NC, _NS, _L = 2, 16, 16
_NW = _NC * _NS
_BPW = _B // _NW
_CHUNK = 128
_NCHUNK = _BPW // _CHUNK
_GPR = _CHUNK // _L

_CBLK = 80000

_mesh = plsc.VectorSubcoreMesh(core_axis_name="c", subcore_axis_name="s")


def _pcalc_body(lb_ref, wt_ref, lwt_ref, mcol_ref, p_ref):
    i = pl.program_id(0)
    lwb = jnp.broadcast_to(lwt_ref[...], (_D, _CBLK))
    s = jnp.sum(wt_ref[...] * lwb, axis=0)
    p_ref[pl.ds(i * _CBLK, _CBLK)] = (
        s * mcol_ref[pl.ds(i * _CBLK, _CBLK)] + lb_ref[0] * (1.0 / _F)
    )


_pcalc = pl.pallas_call(
    _pcalc_body,
    grid=(_T // _CBLK,),
    in_specs=[
        pl.BlockSpec(memory_space=pltpu.SMEM),
        pl.BlockSpec((_D, _CBLK), lambda i: (0, i)),
        pl.BlockSpec((_D, 1), lambda i: (0, 0)),
        pl.BlockSpec((_T,), lambda i: (0,)),
    ],
    out_specs=pl.BlockSpec((_T,), lambda i: (0,)),
    out_shape=jax.ShapeDtypeStruct((_T,), jnp.float32),
)


@functools.partial(
    pl.kernel,
    mesh=_mesh,
    compiler_params=pltpu.CompilerParams(use_tc_tiling_on_sc=False),
    out_type=jax.ShapeDtypeStruct((_B,), jnp.float32),
    scratch_types=[
        pltpu.VMEM((_F, _BPW), jnp.int32),
        pltpu.VMEM((_F, _CHUNK), jnp.int32),
        pltpu.VMEM((_F, _CHUNK), jnp.float32),
        pltpu.VMEM((_CHUNK,), jnp.float32),
        pltpu.SemaphoreType.DMA,
    ],
)
def _pool(xt_hbm, p_hbm, o_hbm, xtv, idxq, sv, hv, sem):
    wid = lax.axis_index("s") * _NC + lax.axis_index("c")
    stages = [
        pltpu.async_copy(
            xt_hbm.at[pl.ds(f * _B + wid * _BPW, _BPW)], xtv.at[f], sem
        )
        for f in range(_F)
    ]
    for cp in stages:
        cp.wait()

    def chunk_body(k, carry):
        base = wid * _BPW + k * _CHUNK
        for f in range(_F):
            for g in range(_GPR):
                idxq[f, pl.ds(g * _L, _L)] = (
                    xtv[f, pl.ds(k * _CHUNK + g * _L, _L)] + f * _V
                )
        copies = [
            pltpu.async_copy(p_hbm.at[idxq.at[f]], sv.at[f], sem)
            for f in range(_F)
        ]
        for cp in copies:
            cp.wait()

        for g in range(_GPR):
            acc = sv[0, pl.ds(g * _L, _L)]
            for f in range(1, _F):
                acc = acc + sv[f, pl.ds(g * _L, _L)]
            hv[pl.ds(g * _L, _L)] = acc

        pltpu.sync_copy(hv, o_hbm.at[pl.ds(base, _CHUNK)])
        return carry

    lax.fori_loop(0, _NCHUNK, chunk_body, 0)


def kernel(x, field_mask, new_field_mask, w, lin_w, lin_b):
    xt = jnp.transpose(x.astype(jnp.int32)).reshape((_F * _B,))
    wt = jnp.transpose(w)
    lwt = jnp.transpose(lin_w)
    mcol = jnp.repeat(field_mask.astype(jnp.float32), _V)
    p = _pcalc(lin_b, wt, lwt, mcol)
    o = _pool(xt, p)
    return o.reshape(_B, 1)


# speedup vs baseline: 6.9691x; 1.0098x over previous
import functools

import jax
import jax.numpy as jnp
from jax import lax
from jax.experimental import pallas as pl
from jax.experimental.pallas import tpu as pltpu
from jax.experimental.pallas import tpu_sc as plsc

_B, _F, _V, _D = 16384, 26, 40000, 16
_T = _F * _V
_NC, _NS, _L = 2, 16, 16
_NW = _NC * _NS
_BPW = _B // _NW
_CHUNK = 512
_NCHUNK = _BPW // _CHUNK
_GPR = _CHUNK // _L

_CBLK = 80000

_mesh = plsc.VectorSubcoreMesh(core_axis_name="c", subcore_axis_name="s")


def _pcalc_body(lb_ref, wt_ref, lwt_ref, mcol_ref, p_ref):
    i = pl.program_id(0)
    lwb = jnp.broadcast_to(lwt_ref[...], (_D, _CBLK))
    s = jnp.sum(wt_ref[...] * lwb, axis=0)
    p_ref[pl.ds(i * _CBLK, _CBLK)] = (
        s * mcol_ref[pl.ds(i * _CBLK, _CBLK)] + lb_ref[0] * (1.0 / _F)
    )


_pcalc = pl.pallas_call(
    _pcalc_body,
    grid=(_T // _CBLK,),
    in_specs=[
        pl.BlockSpec(memory_space=pltpu.SMEM),
        pl.BlockSpec((_D, _CBLK), lambda i: (0, i)),
        pl.BlockSpec((_D, 1), lambda i: (0, 0)),
        pl.BlockSpec((_T,), lambda i: (0,)),
    ],
    out_specs=pl.BlockSpec((_T,), lambda i: (0,)),
    out_shape=jax.ShapeDtypeStruct((_T,), jnp.float32),
)


@functools.partial(
    pl.kernel,
    mesh=_mesh,
    compiler_params=pltpu.CompilerParams(use_tc_tiling_on_sc=False),
    out_type=jax.ShapeDtypeStruct((_B,), jnp.float32),
    scratch_types=[
        pltpu.VMEM((_F, _BPW), jnp.int32),
        pltpu.VMEM((_F, _CHUNK), jnp.int32),
        pltpu.VMEM((_F, _CHUNK), jnp.float32),
        pltpu.VMEM((_CHUNK,), jnp.float32),
        pltpu.SemaphoreType.DMA,
    ],
)
def _pool(xt_hbm, p_hbm, o_hbm, xtv, idxq, sv, hv, sem):
    wid = lax.axis_index("s") * _NC + lax.axis_index("c")
    stages = [
        pltpu.async_copy(
            xt_hbm.at[pl.ds(f * _B + wid * _BPW, _BPW)], xtv.at[f], sem
        )
        for f in range(_F)
    ]
    for cp in stages:
        cp.wait()

    def chunk_body(k, carry):
        base = wid * _BPW + k * _CHUNK
        for f in range(_F):
            for g in range(_GPR):
                idxq[f, pl.ds(g * _L, _L)] = (
                    xtv[f, pl.ds(k * _CHUNK + g * _L, _L)] + f * _V
                )
        copies = [
            pltpu.async_copy(p_hbm.at[idxq.at[f]], sv.at[f], sem)
            for f in range(_F)
        ]
        copies[0].wait()
        acc = [sv[0, pl.ds(g * _L, _L)] for g in range(_GPR)]
        for f in range(1, _F):
            copies[f].wait()
            acc = [a + sv[f, pl.ds(g * _L, _L)] for g, a in enumerate(acc)]
        for g in range(_GPR):
            hv[pl.ds(g * _L, _L)] = acc[g]

        pltpu.sync_copy(hv, o_hbm.at[pl.ds(base, _CHUNK)])
        return carry

    lax.fori_loop(0, _NCHUNK, chunk_body, 0)


def kernel(x, field_mask, new_field_mask, w, lin_w, lin_b):
    xt = jnp.transpose(x.astype(jnp.int32)).reshape((_F * _B,))
    wt = jnp.transpose(w)
    lwt = jnp.transpose(lin_w)
    mcol = jnp.repeat(field_mask.astype(jnp.float32), _V)
    p = _pcalc(lin_b, wt, lwt, mcol)
    o = _pool(xt, p)
    return o.reshape(_B, 1)


# speedup vs baseline: 7.1877x; 1.0314x over previous
import functools

import jax
import jax.numpy as jnp
from jax import lax
from jax.experimental import pallas as pl
from jax.experimental.pallas import tpu as pltpu
from jax.experimental.pallas import tpu_sc as plsc

_B, _F, _V, _D = 16384, 26, 40000, 16
_T = _F * _V
_NC, _NS, _L = 2, 16, 16
_NW = _NC * _NS
_BPW = _B // _NW
_CHUNK = 512
_NCHUNK = _BPW // _CHUNK
_GPR = _CHUNK // _L

_CBLK = 208000

_mesh = plsc.VectorSubcoreMesh(core_axis_name="c", subcore_axis_name="s")


def _pcalc_body(lb_ref, wt_ref, lwt_ref, mcol_ref, p_ref):
    i = pl.program_id(0)
    lwb = jnp.broadcast_to(lwt_ref[...], (_D, _CBLK))
    s = jnp.sum(wt_ref[...] * lwb, axis=0)
    p_ref[pl.ds(i * _CBLK, _CBLK)] = (
        s * mcol_ref[pl.ds(i * _CBLK, _CBLK)] + lb_ref[0] * (1.0 / _F)
    )


_pcalc = pl.pallas_call(
    _pcalc_body,
    grid=(_T // _CBLK,),
    in_specs=[
        pl.BlockSpec(memory_space=pltpu.SMEM),
        pl.BlockSpec((_D, _CBLK), lambda i: (0, i)),
        pl.BlockSpec((_D, 1), lambda i: (0, 0)),
        pl.BlockSpec((_T,), lambda i: (0,)),
    ],
    out_specs=pl.BlockSpec((_T,), lambda i: (0,)),
    out_shape=jax.ShapeDtypeStruct((_T,), jnp.float32),
)


@functools.partial(
    pl.kernel,
    mesh=_mesh,
    compiler_params=pltpu.CompilerParams(use_tc_tiling_on_sc=False),
    out_type=jax.ShapeDtypeStruct((_B,), jnp.float32),
    scratch_types=[
        pltpu.VMEM((_F, _BPW), jnp.int32),
        pltpu.VMEM((_F, _CHUNK), jnp.int32),
        pltpu.VMEM((_F, _CHUNK), jnp.float32),
        pltpu.VMEM((_CHUNK,), jnp.float32),
        pltpu.SemaphoreType.DMA,
    ],
)
def _pool(xt_hbm, p_hbm, o_hbm, xtv, idxq, sv, hv, sem):
    wid = lax.axis_index("s") * _NC + lax.axis_index("c")
    stages = [
        pltpu.async_copy(
            xt_hbm.at[pl.ds(f * _B + wid * _BPW, _BPW)], xtv.at[f], sem
        )
        for f in range(_F)
    ]
    for cp in stages:
        cp.wait()

    def chunk_body(k, carry):
        base = wid * _BPW + k * _CHUNK
        for f in range(_F):
            for g in range(_GPR):
                idxq[f, pl.ds(g * _L, _L)] = (
                    xtv[f, pl.ds(k * _CHUNK + g * _L, _L)] + f * _V
                )
        copies = [
            pltpu.async_copy(p_hbm.at[idxq.at[f]], sv.at[f], sem)
            for f in range(_F)
        ]
        copies[0].wait()
        acc = [sv[0, pl.ds(g * _L, _L)] for g in range(_GPR)]
        for f in range(1, _F):
            copies[f].wait()
            acc = [a + sv[f, pl.ds(g * _L, _L)] for g, a in enumerate(acc)]
        for g in range(_GPR):
            hv[pl.ds(g * _L, _L)] = acc[g]

        pltpu.sync_copy(hv, o_hbm.at[pl.ds(base, _CHUNK)])
        return carry

    lax.fori_loop(0, _NCHUNK, chunk_body, 0)


def kernel(x, field_mask, new_field_mask, w, lin_w, lin_b):
    xt = jnp.transpose(x.astype(jnp.int32)).reshape((_F * _B,))
    wt = jnp.transpose(w)
    lwt = jnp.transpose(lin_w)
    mcol = jnp.repeat(field_mask.astype(jnp.float32), _V)
    p = _pcalc(lin_b, wt, lwt, mcol)
    o = _pool(xt, p)
    return o.reshape(_B, 1)
